# baseline (speedup 1.0000x reference)
import functools

import jax
import jax.numpy as jnp
from jax import lax
from jax.experimental import pallas as pl
from jax.experimental.pallas import tpu as pltpu
from jax.experimental.pallas import tpu_sc as plsc

_BATCH = 1024
_SEQ = 200
_PQ_M = 8
_VALS = 256
_SUB = 16

_N = _BATCH * _SEQ
_NC, _NS = 2, 16
_NW = _NC * _NS
_TOK_W = _N // _NW
_C = 128
_NCHUNK = _TOK_W // _C
_F = _C * _PQ_M

_ITEMS_PAD = 62501


def _body(ids_hbm, codes_hbm, cent_hbm, out_hbm, ids_v, gidx_v, codes_v,
          fidx_v, out_v, sem_c, sem_g, sem_w):
    wid = lax.axis_index("s") * _NC + lax.axis_index("c")
    base_tok = wid * _TOK_W
    lane = jnp.arange(16, dtype=jnp.int32)
    row_pat = lane >> 3
    dim_pat = lane & 7
    off_pat = dim_pat << 8

    def prefetch(c, buf):
        tok0 = base_tok + c * _C
        pltpu.sync_copy(ids_hbm.at[pl.ds(tok0, _C)], ids_v.at[buf])
        for j in range(_C // 16):
            gidx_v[buf, pl.ds(j * 16, 16)] = (
                ids_v[buf, pl.ds(j * 16, 16)] >> 4)
        pltpu.async_copy(codes_hbm.at[gidx_v.at[buf]], codes_v.at[buf],
                         sem_c)

    def wait_codes():
        pltpu.make_async_copy(codes_hbm.at[gidx_v.at[0]], codes_v.at[0],
                              sem_c).wait()

    prefetch(0, 0)

    def chunk(c, _):
        buf = c & 1
        tok0 = base_tok + c * _C
        orow0 = tok0 * _PQ_M

        wait_codes()

        @pl.when(c + 1 < _NCHUNK)
        def _():
            prefetch(c + 1, 1 - buf)

        @pl.when(c >= 2)
        def _():
            pltpu.make_async_copy(
                out_v.at[buf], out_hbm.at[pl.ds(0, _F)],
                sem_w.at[buf],
            ).wait()

        def fidx_row(g, _):
            t16 = 2 * g + row_pat
            ids16 = plsc.load_gather(ids_v.at[buf], [t16])
            cols = ((ids16 & 15) << 3) | dim_pat
            code16 = plsc.load_gather(codes_v.at[buf], [t16, cols])
            fidx_v[pl.ds(g * 16, 16)] = code16 + off_pat
            return _

        lax.fori_loop(0, _F // 16, fidx_row, None, unroll=True)

        pltpu.async_copy(cent_hbm.at[fidx_v], out_v.at[buf], sem_g)
        pltpu.make_async_copy(cent_hbm.at[fidx_v], out_v.at[buf],
                              sem_g).wait()

        pltpu.async_copy(out_v.at[buf], out_hbm.at[pl.ds(orow0, _F)],
                         sem_w.at[buf])
        return _

    lax.fori_loop(0, _NCHUNK, chunk, None)

    for b in range(2):
        pltpu.make_async_copy(out_v.at[b], out_hbm.at[pl.ds(0, _F)],
                              sem_w.at[b]).wait()


@functools.partial(jax.jit, static_argnames=())
def kernel(input_ids, item_codes, centroids):
    ids1 = input_ids.reshape(_N)
    codes_flat = item_codes.reshape(-1)
    codes128 = jnp.concatenate(
        [codes_flat,
         jnp.zeros((_ITEMS_PAD * 128 - codes_flat.shape[0],), jnp.int32)]
    ).reshape(_ITEMS_PAD, 128)
    cent = centroids.reshape(_PQ_M * _VALS, _SUB)
    run = pl.kernel(
        _body,
        out_type=jax.ShapeDtypeStruct((_N * _PQ_M, _SUB), jnp.float32),
        mesh=plsc.VectorSubcoreMesh(
            core_axis_name="c", subcore_axis_name="s",
            num_cores=_NC, num_subcores=_NS,
        ),
        scratch_types=[
            pltpu.VMEM((2, _C), jnp.int32),
            pltpu.VMEM((2, _C), jnp.int32),
            pltpu.VMEM((2, _C, 128), jnp.int32),
            pltpu.VMEM((_F,), jnp.int32),
            pltpu.VMEM((2, _F, _SUB), jnp.float32),
            pltpu.SemaphoreType.DMA,
            pltpu.SemaphoreType.DMA,
            pltpu.SemaphoreType.DMA((2,)),
        ],
        compiler_params=pltpu.CompilerParams(use_tc_tiling_on_sc=False,
                                             needs_layout_passes=False),
    )
    out = run(ids1, codes128, cent)
    return out.reshape(_BATCH, _SEQ, _PQ_M * _SUB)
